# baseline (speedup 1.0000x reference)
import functools

import jax
import jax.numpy as jnp
from jax import lax
from jax.experimental import pallas as pl
from jax.experimental.pallas import tpu as pltpu
from jax.experimental.pallas import tpu_sc as plsc

BATCH = 16384
D = 32
PACK = 4
WIDE = PACK * D
NC = 2
NS = 16
NW = NC * NS
BPW = BATCH // NW
CHUNK = 128
NCH = BPW // CHUNK


def _sc_body(uids_hbm, iids_hbm, utab_hbm, itab_hbm, out_hbm,
             uid_v, iid_v, uq_v, iq_v, ubuf, ibuf, out_v, sem0, sem1):
    wid = lax.axis_index("s") * NC + lax.axis_index("c")
    base = wid * BPW

    pltpu.sync_copy(uids_hbm.at[wid], uid_v)
    pltpu.sync_copy(iids_hbm.at[wid], iid_v)

    def scale(i, carry):
        uq_v[pl.ds(i * 16, 16)] = jax.lax.shift_right_logical(
            uid_v[pl.ds(i * 16, 16)], 2)
        iq_v[pl.ds(i * 16, 16)] = jax.lax.shift_right_logical(
            iid_v[pl.ds(i * 16, 16)], 2)
        return carry
    lax.fori_loop(0, BPW // 16, scale, 0)

    sems = [sem0, sem1]

    def fire(k):
        slot = k % 2
        cu = pltpu.async_copy(
            utab_hbm.at[uq_v.at[pl.ds(k * CHUNK, CHUNK)]],
            ubuf.at[pl.ds(slot * CHUNK, CHUNK)], sems[slot])
        ci = pltpu.async_copy(
            itab_hbm.at[iq_v.at[pl.ds(k * CHUNK, CHUNK)]],
            ibuf.at[pl.ds(slot * CHUNK, CHUNK)], sems[slot])
        return cu, ci

    lane = lax.iota(jnp.int32, 16)
    pend = fire(0)
    for k in range(NCH):
        nxt = fire(k + 1) if k + 1 < NCH else None
        pend[0].wait()
        pend[1].wait()
        slot = k % 2

        def block(blk, carry):
            row = slot * CHUNK + blk * 16 + lane
            ucol = (uid_v[pl.ds(k * CHUNK + blk * 16, 16)] & (PACK - 1)) * D
            icol = (iid_v[pl.ds(k * CHUNK + blk * 16, 16)] & (PACK - 1)) * D
            acc = jnp.zeros((16,), jnp.float32)
            for j in range(D):
                ug = plsc.load_gather(ubuf, [row, ucol + j])
                ig = plsc.load_gather(ibuf, [row, icol + j])
                acc = acc + ug * ig
            out_v[pl.ds(k * CHUNK + blk * 16, 16)] = acc
            return carry

        lax.fori_loop(0, CHUNK // 16, block, 0)
        pend = nxt

    pltpu.sync_copy(out_v, out_hbm.at[pl.ds(base, BPW)])


_sc_call = functools.partial(
    pl.kernel,
    out_type=jax.ShapeDtypeStruct((BATCH,), jnp.float32),
    mesh=plsc.VectorSubcoreMesh(core_axis_name="c", subcore_axis_name="s"),
    compiler_params=pltpu.CompilerParams(
        needs_layout_passes=False, use_tc_tiling_on_sc=True),
    scratch_types=[
        pltpu.VMEM((BPW,), jnp.int32),
        pltpu.VMEM((BPW,), jnp.int32),
        pltpu.VMEM((BPW,), jnp.int32),
        pltpu.VMEM((BPW,), jnp.int32),
        pltpu.VMEM((2 * CHUNK, WIDE), jnp.float32),
        pltpu.VMEM((2 * CHUNK, WIDE), jnp.float32),
        pltpu.VMEM((BPW,), jnp.float32),
        pltpu.SemaphoreType.DMA,
        pltpu.SemaphoreType.DMA,
    ],
)(_sc_body)


def kernel(user_ids, item_ids, user_table, item_table):
    uids = user_ids.reshape(NW, BPW)
    iids = item_ids.reshape(NW, BPW)
    ut = user_table.reshape(-1, WIDE)
    it = item_table.reshape(-1, WIDE)
    out = _sc_call(uids, iids, ut, it)
    return out[:, None]
